# baseline (speedup 1.0000x reference)
import functools

import jax
import jax.numpy as jnp
from jax import lax
from jax.experimental import pallas as pl
from jax.experimental.pallas import tpu as pltpu
from jax.experimental.pallas import tpu_sc as plsc

_TEMP = 0.1
_NSAMP = 10
_H = 224
_W = 224
_HW = _H * _W
_LANES = 128
_CHUNKS = _HW // _LANES
_ROWS = 768
_RPB = 8
_REC = 512
_MLANE = _CHUNKS
_NW = 32
_RPW = _ROWS // _NW


def _dense_body(x_ref, o_ref, f_ref):
    ones_row = jnp.ones((1, _LANES), jnp.float32)
    r392 = lax.broadcasted_iota(jnp.int32, (_CHUNKS, _CHUNKS), 0)
    c392 = lax.broadcasted_iota(jnp.int32, (_CHUNKS, _CHUNKS), 1)
    tri392 = (r392 <= c392).astype(jnp.float32)
    lts392 = (c392 < r392).astype(jnp.float32)
    r128 = lax.broadcasted_iota(jnp.int32, (_LANES, _LANES), 0)
    c128 = lax.broadcasted_iota(jnp.int32, (_LANES, _LANES), 1)
    tri128 = (r128 <= c128).astype(jnp.float32)
    for r in range(_RPB):
        x = x_ref[r]
        lg = x / jnp.float32(_TEMP)
        m = jnp.max(lg)
        e = jnp.exp(lg - m)
        srow = lax.dot_general(ones_row, e, (((1,), (1,)), ((), ())),
                               precision=lax.Precision.HIGHEST,
                               preferred_element_type=jnp.float32)
        p = lax.dot_general(srow, tri392, (((1,), (0,)), ((), ())),
                            precision=lax.Precision.HIGHEST,
                            preferred_element_type=jnp.float32)
        mv = jnp.full((1, 8), m, jnp.float32)
        pad = jnp.zeros((1, _REC - _CHUNKS - 8), jnp.float32)
        o_ref[r] = jnp.concatenate([p, mv, pad], axis=1)
        pexc_col = lax.dot_general(lts392, srow, (((1,), (1,)), ((), ())),
                                   precision=lax.Precision.HIGHEST,
                                   preferred_element_type=jnp.float32)
        e_hi = e.astype(jnp.bfloat16).astype(jnp.float32)
        e_lo = e - e_hi
        c_hi = lax.dot_general(e_hi, tri128, (((1,), (0,)), ((), ())),
                               preferred_element_type=jnp.float32)
        c_lo = lax.dot_general(e_lo, tri128, (((1,), (0,)), ((), ())),
                               preferred_element_type=jnp.float32)
        c_loc = c_hi + c_lo
        f_ref[r] = c_loc + pexc_col


_GB = 8


def _sc_body(pm_hbm, u_hbm, f_hbm, o_hbm,
             pm_v, u_v, idx_v, chunk_v, o_v, sem):
    cid = lax.axis_index("c")
    sid = lax.axis_index("s")
    wid = sid * 2 + cid
    base = wid * _RPW
    pltpu.sync_copy(pm_hbm.at[pl.ds(base, _RPW)], pm_v)
    pltpu.sync_copy(u_hbm.at[pl.ds(base, _RPW)], u_v)
    lane = lax.iota(jnp.int32, 16)
    live = lane < _NSAMP
    for b in range(_RPW // _GB):
        nfs = []
        for rr in range(_GB):
            k = b * _GB + rr
            uu = u_v[k, :]
            krow = (uu * 0.0).astype(jnp.int32) + k
            z = plsc.load_gather(pm_v, [krow, lane * 0 + (_CHUNKS - 1)])
            t = uu * z
            lo = jnp.zeros((16,), jnp.int32)
            hi = jnp.full((16,), _CHUNKS, jnp.int32)
            for _ in range(9):
                mid = lax.shift_right_arithmetic(lo + hi, 1)
                pmid = plsc.load_gather(pm_v, [krow, mid])
                cond = pmid < t
                lo = jnp.where(cond, mid + 1, lo)
                hi = jnp.where(cond, hi, mid)
            nfs.append((lo, t))
            idx_v[pl.ds(rr * 16, 16)] = (base + k) * _CHUNKS + lo
        pltpu.async_copy(f_hbm.at[idx_v], chunk_v, sem).wait()
        for rr in range(_GB):
            k = b * _GB + rr
            nf, t = nfs[rr]
            lo2 = jnp.zeros((16,), jnp.int32)
            hi2 = jnp.full((16,), _LANES, jnp.int32)
            for _ in range(7):
                mid2 = lax.shift_right_arithmetic(lo2 + hi2, 1)
                fv = plsc.load_gather(chunk_v, [rr * 16 + lane, mid2])
                cond2 = fv < t
                lo2 = jnp.where(cond2, mid2 + 1, lo2)
                hi2 = jnp.where(cond2, hi2, mid2)
            idx = jnp.minimum(nf * _LANES + lo2, _HW - 1)
            xq = (idx % _W).astype(jnp.float32)
            yq = (idx // _W).astype(jnp.float32)
            px = jnp.sum(jnp.where(live, xq, 0.0)) * jnp.float32(
                1.0 / (_W * _NSAMP))
            py = jnp.sum(jnp.where(live, yq, 0.0)) * jnp.float32(
                1.0 / (_H * _NSAMP))
            o_v[k, :] = jnp.where(lane == 0, px, jnp.where(lane == 1, py, 0.0))
    pltpu.sync_copy(o_v, o_hbm.at[pl.ds(base, _RPW)])


_sc_sample_cache = []


def _get_sc_sample():
    if not _sc_sample_cache:
        _sc_sample_cache.append(functools.partial(
            pl.kernel,
            mesh=plsc.VectorSubcoreMesh(
                core_axis_name="c", subcore_axis_name="s"),
            compiler_params=pltpu.CompilerParams(needs_layout_passes=False),
            out_type=jax.ShapeDtypeStruct((_ROWS, 16), jnp.float32),
            scratch_types=[
                pltpu.VMEM((_RPW, _REC), jnp.float32),
                pltpu.VMEM((_RPW, 16), jnp.float32),
                pltpu.VMEM((_GB * 16,), jnp.int32),
                pltpu.VMEM((_GB * 16, _LANES), jnp.float32),
                pltpu.VMEM((_RPW, 16), jnp.float32),
                pltpu.SemaphoreType.DMA,
            ],
        )(_sc_body))
    return _sc_sample_cache[0]


def kernel(heatmap):
    B, C, H, W = heatmap.shape
    n = B * C
    hm3 = heatmap.reshape(n, _CHUNKS, _LANES)
    pm, f = pl.pallas_call(
        _dense_body,
        grid=(n // _RPB,),
        in_specs=[pl.BlockSpec((_RPB, _CHUNKS, _LANES), lambda i: (i, 0, 0))],
        out_specs=[
            pl.BlockSpec((_RPB, 1, _REC), lambda i: (i, 0, 0)),
            pl.BlockSpec((_RPB, _CHUNKS, _LANES), lambda i: (i, 0, 0)),
        ],
        out_shape=[
            jax.ShapeDtypeStruct((n, 1, _REC), jnp.float32),
            jax.ShapeDtypeStruct((n, _CHUNKS, _LANES), jnp.float32),
        ],
    )(hm3)
    pm2 = pm.reshape(n, _REC)
    skey = jax.random.fold_in(jax.random.key(0), 1)
    u = jax.random.uniform(skey, (n, _NSAMP), dtype=heatmap.dtype)
    u2 = jnp.concatenate(
        [u, jnp.full((n, 16 - _NSAMP), 0.5, heatmap.dtype)], axis=1)
    f2 = f.reshape(n * _CHUNKS, _LANES)
    o = _get_sc_sample()(pm2, u2, f2)
    return o[:, :2].reshape(B, C, 2)
